# baseline (speedup 1.0000x reference)
import functools

import jax
import jax.numpy as jnp
from jax import lax
from jax.experimental import pallas as pl
from jax.experimental.pallas import tpu as pltpu
from jax.experimental.pallas import tpu_sc as plsc

_NC, _NS, _L = 2, 16, 16
_NW = _NC * _NS
_CHUNK = 8
_NSLOT = 3
_UNROLL = 16


_SKEW = 24


@functools.cache
def _build_sc_kernel(B, S, D, dtype):
    rows_per_w = S // _NW
    pair_rows = _NC * rows_per_w
    r0 = rows_per_w + _SKEW
    r1 = pair_rows - r0
    n0, n1 = r0 // _CHUNK, r1 // _CHUNK
    vecs_per_row = D // _L

    mesh = plsc.VectorSubcoreMesh(core_axis_name="c", subcore_axis_name="s")

    @functools.partial(
        pl.kernel,
        out_type=jax.ShapeDtypeStruct((B, S, D), dtype),
        mesh=mesh,
        scratch_types=[
            pltpu.VMEM((_NSLOT, _CHUNK, D), jnp.float32),
            pltpu.VMEM((_NSLOT, B, _CHUNK, D), jnp.float32),
            pltpu.SemaphoreType.DMA((_NSLOT,)),
            pltpu.SemaphoreType.DMA((_NSLOT,)),
        ],
    )
    def sc_kernel(in_hbm, emb_hbm, out_hbm, emb_v, io_v, lsem, ssem):
        c = lax.axis_index("c")
        s = lax.axis_index("s")
        base = s * pair_rows + c * r0
        n_chunks = jnp.where(c == 0, n0, n1)

        def load_descs(ci, slot):
            row0 = base + ci * _CHUNK
            return [
                pltpu.make_async_copy(emb_hbm.at[pl.ds(row0, _CHUNK)],
                                      emb_v.at[slot], lsem.at[slot]),
                pltpu.make_async_copy(in_hbm.at[:, pl.ds(row0, _CHUNK)],
                                      io_v.at[slot], lsem.at[slot]),
            ]

        def store_descs(ci, slot):
            row0 = base + ci * _CHUNK
            return [pltpu.make_async_copy(io_v.at[slot],
                                          out_hbm.at[:, pl.ds(row0, _CHUNK)],
                                          ssem.at[slot])]

        def start_loads(ci, slot):
            for d in load_descs(ci, slot):
                d.start()

        def wait_loads(ci, slot):
            for d in load_descs(ci, slot):
                d.wait()

        def start_stores(ci, slot):
            for d in store_descs(ci, slot):
                d.start()

        def wait_stores(ci, slot):
            for d in store_descs(ci, slot):
                d.wait()

        start_loads(0, 0)

        @pl.loop(0, n_chunks)
        def _chunk(ci):
            slot = lax.rem(ci, _NSLOT)
            nxt = lax.rem(ci + 1, _NSLOT)

            @pl.when(ci + 1 < n_chunks)
            def _():
                @pl.when(ci >= _NSLOT - 1)
                def _():
                    wait_stores(ci - (_NSLOT - 1), nxt)
                start_loads(ci + 1, nxt)

            wait_loads(ci, slot)

            @pl.loop(0, _CHUNK)
            def _row(r):
                @plsc.parallel_loop(0, vecs_per_row, unroll=_UNROLL)
                def _col(cv):
                    sl = pl.ds(cv * _L, _L)
                    e = emb_v[slot, r, sl]
                    for b in range(B):
                        plsc.addupdate(io_v.at[slot, b, r, sl], e)

            start_stores(ci, slot)

        for k in range(_NSLOT, 0, -1):
            tail = n_chunks - k
            wait_stores(tail, lax.rem(tail, _NSLOT))

    return sc_kernel


def kernel(inputs, embeddings):
    B, S, D = inputs.shape
    pos = embeddings[:S]
    return _build_sc_kernel(B, S, D, inputs.dtype)(inputs, pos)


# speedup vs baseline: 1.0081x; 1.0081x over previous
import functools

import jax
import jax.numpy as jnp
from jax import lax
from jax.experimental import pallas as pl
from jax.experimental.pallas import tpu as pltpu
from jax.experimental.pallas import tpu_sc as plsc

_NC, _NS, _L = 2, 16, 16
_NW = _NC * _NS
_CHUNK = 8
_NSLOT = 3
_UNROLL = 16


_SKEW = -24


@functools.cache
def _build_sc_kernel(B, S, D, dtype):
    rows_per_w = S // _NW
    pair_rows = _NC * rows_per_w
    r0 = rows_per_w + _SKEW
    r1 = pair_rows - r0
    n0, n1 = r0 // _CHUNK, r1 // _CHUNK
    vecs_per_row = D // _L

    mesh = plsc.VectorSubcoreMesh(core_axis_name="c", subcore_axis_name="s")

    @functools.partial(
        pl.kernel,
        out_type=jax.ShapeDtypeStruct((B, S, D), dtype),
        mesh=mesh,
        scratch_types=[
            pltpu.VMEM((_NSLOT, _CHUNK, D), jnp.float32),
            pltpu.VMEM((_NSLOT, B, _CHUNK, D), jnp.float32),
            pltpu.SemaphoreType.DMA((_NSLOT,)),
            pltpu.SemaphoreType.DMA((_NSLOT,)),
        ],
    )
    def sc_kernel(in_hbm, emb_hbm, out_hbm, emb_v, io_v, lsem, ssem):
        c = lax.axis_index("c")
        s = lax.axis_index("s")
        base = s * pair_rows + c * r0
        n_chunks = jnp.where(c == 0, n0, n1)

        def load_descs(ci, slot):
            row0 = base + ci * _CHUNK
            return [
                pltpu.make_async_copy(emb_hbm.at[pl.ds(row0, _CHUNK)],
                                      emb_v.at[slot], lsem.at[slot]),
                pltpu.make_async_copy(in_hbm.at[:, pl.ds(row0, _CHUNK)],
                                      io_v.at[slot], lsem.at[slot]),
            ]

        def store_descs(ci, slot):
            row0 = base + ci * _CHUNK
            return [pltpu.make_async_copy(io_v.at[slot],
                                          out_hbm.at[:, pl.ds(row0, _CHUNK)],
                                          ssem.at[slot])]

        def start_loads(ci, slot):
            for d in load_descs(ci, slot):
                d.start()

        def wait_loads(ci, slot):
            for d in load_descs(ci, slot):
                d.wait()

        def start_stores(ci, slot):
            for d in store_descs(ci, slot):
                d.start()

        def wait_stores(ci, slot):
            for d in store_descs(ci, slot):
                d.wait()

        start_loads(0, 0)

        @pl.loop(0, n_chunks)
        def _chunk(ci):
            slot = lax.rem(ci, _NSLOT)
            nxt = lax.rem(ci + 1, _NSLOT)

            @pl.when(ci + 1 < n_chunks)
            def _():
                @pl.when(ci >= _NSLOT - 1)
                def _():
                    wait_stores(ci - (_NSLOT - 1), nxt)
                start_loads(ci + 1, nxt)

            wait_loads(ci, slot)

            @pl.loop(0, _CHUNK)
            def _row(r):
                @plsc.parallel_loop(0, vecs_per_row, unroll=_UNROLL)
                def _col(cv):
                    sl = pl.ds(cv * _L, _L)
                    e = emb_v[slot, r, sl]
                    for b in range(B):
                        plsc.addupdate(io_v.at[slot, b, r, sl], e)

            start_stores(ci, slot)

        for k in range(_NSLOT, 0, -1):
            tail = n_chunks - k
            wait_stores(tail, lax.rem(tail, _NSLOT))

    return sc_kernel


def kernel(inputs, embeddings):
    B, S, D = inputs.shape
    pos = embeddings[:S]
    return _build_sc_kernel(B, S, D, inputs.dtype)(inputs, pos)


# speedup vs baseline: 1.0352x; 1.0269x over previous
import functools

import jax
import jax.numpy as jnp
from jax import lax
from jax.experimental import pallas as pl
from jax.experimental.pallas import tpu as pltpu
from jax.experimental.pallas import tpu_sc as plsc

_NC, _NS, _L = 2, 16, 16
_NW = _NC * _NS
_CHUNK = 8
_NSLOT = 3
_UNROLL = 16


_SKEW = 0


@functools.cache
def _build_sc_kernel(B, S, D, dtype):
    rows_per_w = S // _NW
    pair_rows = _NC * rows_per_w
    r0 = rows_per_w + _SKEW
    r1 = pair_rows - r0
    n0, n1 = r0 // _CHUNK, r1 // _CHUNK
    vecs_per_row = D // _L

    mesh = plsc.VectorSubcoreMesh(core_axis_name="c", subcore_axis_name="s")

    @functools.partial(
        pl.kernel,
        out_type=jax.ShapeDtypeStruct((B, S, D), dtype),
        mesh=mesh,
        scratch_types=[
            pltpu.VMEM((_NSLOT, _CHUNK, D), jnp.float32),
            pltpu.VMEM((_NSLOT, B, _CHUNK, D), jnp.float32),
            pltpu.SemaphoreType.DMA((_NSLOT,)),
            pltpu.SemaphoreType.DMA((_NSLOT,)),
        ],
    )
    def sc_kernel(in_hbm, emb_hbm, out_hbm, emb_v, io_v, lsem, ssem):
        c = lax.axis_index("c")
        s = lax.axis_index("s")
        base = s * pair_rows + c * r0
        n_chunks = jnp.where(c == 0, n0, n1)

        def load_descs(ci, slot):
            row0 = base + ci * _CHUNK
            return [
                pltpu.make_async_copy(emb_hbm.at[pl.ds(row0, _CHUNK)],
                                      emb_v.at[slot], lsem.at[slot]),
                pltpu.make_async_copy(in_hbm.at[:, pl.ds(row0, _CHUNK)],
                                      io_v.at[slot], lsem.at[slot]),
            ]

        def store_descs(ci, slot):
            row0 = base + ci * _CHUNK
            return [pltpu.make_async_copy(io_v.at[slot],
                                          out_hbm.at[:, pl.ds(row0, _CHUNK)],
                                          ssem.at[slot])]

        def start_loads(ci, slot):
            for d in load_descs(ci, slot):
                d.start()

        def wait_loads(ci, slot):
            for d in load_descs(ci, slot):
                d.wait()

        def start_stores(ci, slot):
            for d in store_descs(ci, slot):
                d.start()

        def wait_stores(ci, slot):
            for d in store_descs(ci, slot):
                d.wait()

        start_loads(0, 0)

        @pl.loop(0, n_chunks)
        def _chunk(ci):
            slot = lax.rem(ci, _NSLOT)
            nxt = lax.rem(ci + 1, _NSLOT)

            @pl.when(ci + 1 < n_chunks)
            def _():
                @pl.when(ci >= _NSLOT - 1)
                def _():
                    wait_stores(ci - (_NSLOT - 1), nxt)
                start_loads(ci + 1, nxt)

            wait_loads(ci, slot)

            @pl.loop(0, _CHUNK)
            def _row(r):
                @plsc.parallel_loop(0, vecs_per_row, unroll=_UNROLL)
                def _col(cv):
                    sl = pl.ds(cv * _L, _L)
                    e = emb_v[slot, r, sl]
                    for b in range(B):
                        plsc.addupdate(io_v.at[slot, b, r, sl], e)

            start_stores(ci, slot)

        for k in range(_NSLOT, 0, -1):
            tail = n_chunks - k
            wait_stores(tail, lax.rem(tail, _NSLOT))

    return sc_kernel


def kernel(inputs, embeddings):
    B, S, D = inputs.shape
    pos = embeddings[:S]
    return _build_sc_kernel(B, S, D, inputs.dtype)(inputs, pos)
